# baseline (speedup 1.0000x reference)
import dataclasses
import functools

import jax
import jax.numpy as jnp
from jax import lax
from jax.experimental import pallas as pl
from jax.experimental.pallas import tpu as pltpu
from jax.experimental.pallas import tpu_sc as plsc

N = 10000
E = 320000
D_IN = 128
D = 384
C = 40
QW = 96
DV = 128
ROWB = 400
SCALE = 1.0 / (D ** 0.5)
PREC = lax.Precision.HIGHEST


def _dot(a, b):
    return jnp.dot(a, b, preferred_element_type=jnp.float32, precision=PREC)


def _split_vaug(qkv, ones):
    q = qkv[:, :D] * SCALE
    k = qkv[:, D:2 * D]
    v = qkv[:, 2 * D:]
    pad = jnp.zeros_like(ones)
    vq = [jnp.concatenate([v[:, QW * i:QW * (i + 1)], ones, pad], axis=1)
          for i in range(4)]
    return q, k, jnp.stack(vq, axis=0)


def _embed_qkv_kernel(x_ref, we_ref, be_ref, wq_ref, bq_ref,
                      qs_ref, k_ref, vaug_ref):
    h = _dot(x_ref[...], we_ref[...]) + be_ref[...]
    qkv = _dot(h, wq_ref[...]) + bq_ref[...]
    ones = jnp.ones((x_ref.shape[0], 16), jnp.float32)
    q, k, vaug = _split_vaug(qkv, ones)
    qs_ref[...] = q
    k_ref[...] = k
    vaug_ref[...] = vaug


def _embed_qkv(x, W_embed, b_embed, Wqkv, bqkv):
    grid = (N // ROWB,)
    return pl.pallas_call(
        _embed_qkv_kernel,
        grid=grid,
        in_specs=[
            pl.BlockSpec((ROWB, D_IN), lambda i: (i, 0)),
            pl.BlockSpec((D_IN, D), lambda i: (0, 0)),
            pl.BlockSpec((1, D), lambda i: (0, 0)),
            pl.BlockSpec((D, 3 * D), lambda i: (0, 0)),
            pl.BlockSpec((1, 3 * D), lambda i: (0, 0)),
        ],
        out_specs=[
            pl.BlockSpec((ROWB, D), lambda i: (i, 0)),
            pl.BlockSpec((ROWB, D), lambda i: (i, 0)),
            pl.BlockSpec((4, ROWB, DV), lambda i: (0, i, 0)),
        ],
        out_shape=[
            jax.ShapeDtypeStruct((N, D), jnp.float32),
            jax.ShapeDtypeStruct((N, D), jnp.float32),
            jax.ShapeDtypeStruct((4, N, DV), jnp.float32),
        ],
    )(x, W_embed, b_embed.reshape(1, D), Wqkv, bqkv.reshape(1, 3 * D))


def _scores_kernel(q_ref, k_ref, s_ref):
    s_ref[0, 0] = lax.dot_general(
        q_ref[...], k_ref[...], (((1,), (1,)), ((), ())),
        preferred_element_type=jnp.float32, precision=lax.Precision.DEFAULT)


SBM = 1024
SBN = 128
SGM = (N + SBM - 1) // SBM
SGN = (N + SBN - 1) // SBN


def _scores(qs, k):
    return pl.pallas_call(
        _scores_kernel,
        grid=(SGM, SGN),
        in_specs=[
            pl.BlockSpec((SBM, D), lambda i, j: (i, 0)),
            pl.BlockSpec((SBN, D), lambda i, j: (j, 0)),
        ],
        out_specs=pl.BlockSpec((1, 1, SBM, SBN), lambda i, j: (i, j, 0, 0)),
        out_shape=jax.ShapeDtypeStruct((SGM, SGN, SBM, SBN), jnp.float32),
    )(qs, k)


def _agg_from_acc(acc):
    den = acc[0][:, QW:QW + 1]
    agg = jnp.concatenate([acc[i][:, :QW] for i in range(4)], axis=1)
    return agg / (den + 1e-16)


def _proj_qkv_kernel(acc_ref, wo_ref, bo_ref, wq_ref, bq_ref,
                     qs_ref, k_ref, vaug_ref):
    agg = _agg_from_acc(acc_ref)
    t = _dot(agg, wo_ref[...]) + bo_ref[...]
    t = jnp.maximum(t, 0.0)
    qkv = _dot(t, wq_ref[...]) + bq_ref[...]
    ones = jnp.ones((acc_ref.shape[1], 16), jnp.float32)
    q, k, vaug = _split_vaug(qkv, ones)
    qs_ref[...] = q
    k_ref[...] = k
    vaug_ref[...] = vaug


def _proj_qkv(acc, Wo, bo, Wqkv, bqkv):
    grid = (N // ROWB,)
    return pl.pallas_call(
        _proj_qkv_kernel,
        grid=grid,
        in_specs=[
            pl.BlockSpec((4, ROWB, DV), lambda i: (0, i, 0)),
            pl.BlockSpec((D, D), lambda i: (0, 0)),
            pl.BlockSpec((1, D), lambda i: (0, 0)),
            pl.BlockSpec((D, 3 * D), lambda i: (0, 0)),
            pl.BlockSpec((1, 3 * D), lambda i: (0, 0)),
        ],
        out_specs=[
            pl.BlockSpec((ROWB, D), lambda i: (i, 0)),
            pl.BlockSpec((ROWB, D), lambda i: (i, 0)),
            pl.BlockSpec((4, ROWB, DV), lambda i: (0, i, 0)),
        ],
        out_shape=[
            jax.ShapeDtypeStruct((N, D), jnp.float32),
            jax.ShapeDtypeStruct((N, D), jnp.float32),
            jax.ShapeDtypeStruct((4, N, DV), jnp.float32),
        ],
    )(acc, Wo, bo.reshape(1, D), Wqkv, bqkv.reshape(1, 3 * D))


def _final_kernel(acc_ref, wo_ref, bo_ref, wl_ref, bl_ref, o_ref):
    agg = _agg_from_acc(acc_ref)
    t = _dot(agg, wo_ref[...]) + bo_ref[...]
    logits = _dot(t, wl_ref[...]) + bl_ref[...]
    m = jnp.max(logits, axis=1, keepdims=True)
    lse = jnp.log(jnp.sum(jnp.exp(logits - m), axis=1, keepdims=True)) + m
    o_ref[...] = logits - lse


def _final(acc, Wo, bo, W_lin, b_lin):
    grid = (N // ROWB,)
    return pl.pallas_call(
        _final_kernel,
        grid=grid,
        in_specs=[
            pl.BlockSpec((4, ROWB, DV), lambda i: (0, i, 0)),
            pl.BlockSpec((D, D), lambda i: (0, 0)),
            pl.BlockSpec((1, D), lambda i: (0, 0)),
            pl.BlockSpec((D, C), lambda i: (0, 0)),
            pl.BlockSpec((1, C), lambda i: (0, 0)),
        ],
        out_specs=pl.BlockSpec((ROWB, C), lambda i: (i, 0)),
        out_shape=jax.ShapeDtypeStruct((N, C), jnp.float32),
    )(acc, Wo, bo.reshape(1, D), W_lin, b_lin.reshape(1, C))


NSC = 2
NSUB = 16
LANES = 16
CHUNK = 80
EPC = E // NSUB
NCHUNK = EPC // CHUNK
STRIPE = 624
TAIL = N - NSUB * STRIPE


def _edge_phase_sc(Sflat, vtab, src2d, dst2, sidx2, zz):
    mesh = plsc.VectorSubcoreMesh(core_axis_name="c", subcore_axis_name="s")
    cp = pltpu.CompilerParams()
    if "needs_layout_passes" in pltpu.CompilerParams.__dataclass_fields__:
        cp = dataclasses.replace(cp, needs_layout_passes=False)
    if "use_tc_tiling_on_sc" in pltpu.CompilerParams.__dataclass_fields__:
        cp = dataclasses.replace(cp, use_tc_tiling_on_sc=False)

    @functools.partial(
        pl.kernel, mesh=mesh, compiler_params=cp,
        out_type=jax.ShapeDtypeStruct((4 * N, DV), jnp.float32),
        scratch_types=[
            pltpu.VMEM((NCHUNK, CHUNK), jnp.int32),
            pltpu.VMEM((1, CHUNK), jnp.int32),
            pltpu.VMEM((1, CHUNK), jnp.int32),
            pltpu.VMEM((1, CHUNK), jnp.int32),
            pltpu.VMEM((1, CHUNK), jnp.int32),
            pltpu.VMEM((CHUNK,), jnp.float32),
            pltpu.VMEM((CHUNK,), jnp.float32),
            pltpu.VMEM((CHUNK, DV), jnp.float32),
            pltpu.VMEM((CHUNK, DV), jnp.float32),
            pltpu.VMEM_SHARED((N, DV), jnp.float32),
            pltpu.SemaphoreType.DMA,
            pltpu.SemaphoreType.DMA,
            pltpu.SemaphoreType.DMA,
            pltpu.SemaphoreType.DMA,
            pltpu.SemaphoreType.DMA,
            pltpu.SemaphoreType.DMA,
        ],
    )
    def ker(sflat_hbm, vtab_hbm, src2d_hbm, dst2_hbm, sidx2_hbm, zz_hbm,
            out_hbm, dst_v, sia, sib, sra, srb, wa, wb, rowsa, rowsb, acc,
            isa_, isb_, gsa, gsb, ssa, ssb):
        core = lax.axis_index("c")
        sid = lax.axis_index("s")

        pltpu.sync_copy(dst2_hbm.at[pl.ds(sid * NCHUNK, NCHUNK)], dst_v)

        NT = NCHUNK // 2
        for p in range(2):
            q = 2 * core + p
            sbase = sid * NCHUNK
            rbase = (q * NSUB + sid) * NCHUNK

            def issue_idx(si_buf, sr_buf, isem, i):
                pltpu.async_copy(sidx2_hbm.at[pl.ds(sbase + i, 1)],
                                 si_buf, isem)
                pltpu.async_copy(src2d_hbm.at[pl.ds(rbase + i, 1)],
                                 sr_buf, isem)

            def wait_idx(si_buf, sr_buf, isem, i):
                pltpu.make_async_copy(sidx2_hbm.at[pl.ds(sbase + i, 1)],
                                      si_buf, isem).wait()
                pltpu.make_async_copy(src2d_hbm.at[pl.ds(rbase + i, 1)],
                                      sr_buf, isem).wait()

            def issue_gathers(si_buf, sr_buf, w_buf, rows_buf, gsem):
                pltpu.async_copy(sflat_hbm.at[si_buf.at[0]], w_buf, gsem)
                pltpu.async_copy(vtab_hbm.at[sr_buf.at[0]], rows_buf, gsem)

            def wait_gathers(si_buf, sr_buf, w_buf, rows_buf, gsem):
                pltpu.make_async_copy(sflat_hbm.at[si_buf.at[0]], w_buf,
                                      gsem).wait()
                pltpu.make_async_copy(vtab_hbm.at[sr_buf.at[0]], rows_buf,
                                      gsem).wait()

            def compute(w_buf, rows_buf):
                for c in range(CHUNK // LANES):
                    sl = pl.ds(c * LANES, LANES)
                    w_buf[sl] = jnp.exp(w_buf[sl])

                @pl.loop(0, CHUNK)
                def _(j):
                    wspl = plsc.load_gather(
                        w_buf, [jnp.full((LANES,), j, jnp.int32)])
                    for c in range(DV // LANES):
                        sl = pl.ds(c * LANES, LANES)
                        rows_buf[j, sl] = rows_buf[j, sl] * wspl

            def issue_scatter(rows_buf, ssem, i):
                pltpu.async_copy(rows_buf, acc.at[dst_v.at[i]], ssem,
                                 add=True)

            def wait_scatter(rows_buf, ssem, i):
                pltpu.make_async_copy(rows_buf, acc.at[dst_v.at[i]],
                                      ssem).wait()

            pltpu.sync_copy(zz_hbm, acc.at[pl.ds(sid * STRIPE, STRIPE)])

            @pl.when(sid == NSUB - 1)
            def _():
                pltpu.sync_copy(zz_hbm.at[pl.ds(0, TAIL)],
                                acc.at[pl.ds(NSUB * STRIPE, TAIL)])

            plsc.subcore_barrier()

            issue_idx(sia, sra, isa_, 0)
            issue_idx(sib, srb, isb_, 1)
            wait_idx(sia, sra, isa_, 0)
            issue_gathers(sia, sra, wa, rowsa, gsa)

            @pl.loop(0, NT)
            def _(t):
                i0 = 2 * t
                i1 = i0 + 1

                @pl.when(t > 0)
                def _():
                    wait_scatter(rowsb, ssb, i1 - 2)

                wait_idx(sib, srb, isb_, i1)
                issue_gathers(sib, srb, wb, rowsb, gsb)
                wait_gathers(sia, sra, wa, rowsa, gsa)

                @pl.when(t < NT - 1)
                def _():
                    issue_idx(sia, sra, isa_, i0 + 2)

                compute(wa, rowsa)
                issue_scatter(rowsa, ssa, i0)

                @pl.when(t < NT - 1)
                def _():
                    wait_scatter(rowsa, ssa, i0)
                    wait_idx(sia, sra, isa_, i0 + 2)
                    issue_gathers(sia, sra, wa, rowsa, gsa)

                wait_gathers(sib, srb, wb, rowsb, gsb)

                @pl.when(t < NT - 1)
                def _():
                    issue_idx(sib, srb, isb_, i1 + 2)

                compute(wb, rowsb)
                issue_scatter(rowsb, ssb, i1)

            wait_scatter(rowsa, ssa, NCHUNK - 2)
            wait_scatter(rowsb, ssb, NCHUNK - 1)
            plsc.subcore_barrier()
            pltpu.sync_copy(acc.at[pl.ds(sid * STRIPE, STRIPE)],
                            out_hbm.at[pl.ds(q * N + sid * STRIPE, STRIPE)])

            @pl.when(sid == NSUB - 1)
            def _():
                pltpu.sync_copy(
                    acc.at[pl.ds(NSUB * STRIPE, TAIL)],
                    out_hbm.at[pl.ds(q * N + NSUB * STRIPE, TAIL)])

    return ker(Sflat, vtab, src2d, dst2, sidx2, zz)


def _edge_phase(S, vaug, src2d, dst2, sidx2, zz):
    acc = _edge_phase_sc(S.reshape(-1), vaug.reshape(4 * N, DV),
                         src2d, dst2, sidx2, zz)
    return acc.reshape(4, N, DV)


def kernel(x, edge_index, W_embed, b_embed, Wqkv1, bqkv1, Wo1, bo1,
           Wqkv2, bqkv2, Wo2, bo2, W_lin, b_lin):
    src = edge_index[0].astype(jnp.int32)
    dst = edge_index[1].astype(jnp.int32)
    sidx = (((dst >> 10) * SGN + (src >> 7)) << 17) \
        + ((dst & (SBM - 1)) << 7) + (src & (SBN - 1))
    src4 = jnp.concatenate([src, src + N, src + 2 * N, src + 3 * N])
    src2d = src4.reshape(-1, CHUNK)
    dst2 = dst.reshape(-1, CHUNK)
    sidx2 = sidx.reshape(-1, CHUNK)
    zz = jnp.zeros((STRIPE, DV), jnp.float32)

    qs, k, vaug = _embed_qkv(x, W_embed, b_embed, Wqkv1, bqkv1)
    S = _scores(qs, k)
    acc = _edge_phase(S, vaug, src2d, dst2, sidx2, zz)
    qs2, k2, vaug2 = _proj_qkv(acc, Wo1, bo1, Wqkv2, bqkv2)
    S2 = _scores(qs2, k2)
    acc2 = _edge_phase(S2, vaug2, src2d, dst2, sidx2, zz)
    return _final(acc2, Wo2, bo2, W_lin, b_lin)


# speedup vs baseline: 1.0915x; 1.0915x over previous
import dataclasses
import functools

import jax
import jax.numpy as jnp
from jax import lax
from jax.experimental import pallas as pl
from jax.experimental.pallas import tpu as pltpu
from jax.experimental.pallas import tpu_sc as plsc

N = 10000
E = 320000
D_IN = 128
D = 384
C = 40
QW = 96
DV = 128
ROWB = 400
SCALE = 1.0 / (D ** 0.5)
PREC = lax.Precision.HIGHEST


def _dot(a, b):
    return jnp.dot(a, b, preferred_element_type=jnp.float32, precision=PREC)


def _split_vaug(qkv, ones):
    q = qkv[:, :D] * SCALE
    k = qkv[:, D:2 * D]
    v = qkv[:, 2 * D:]
    pad = jnp.zeros_like(ones)
    vq = [jnp.concatenate([v[:, QW * i:QW * (i + 1)], ones, pad], axis=1)
          for i in range(4)]
    return q, k, jnp.stack(vq, axis=0)


def _embed_qkv_kernel(x_ref, we_ref, be_ref, wq_ref, bq_ref,
                      qs_ref, k_ref, vaug_ref):
    h = _dot(x_ref[...], we_ref[...]) + be_ref[...]
    qkv = _dot(h, wq_ref[...]) + bq_ref[...]
    ones = jnp.ones((x_ref.shape[0], 16), jnp.float32)
    q, k, vaug = _split_vaug(qkv, ones)
    qs_ref[...] = q
    k_ref[...] = k
    vaug_ref[...] = vaug


def _embed_qkv(x, W_embed, b_embed, Wqkv, bqkv):
    grid = (N // ROWB,)
    return pl.pallas_call(
        _embed_qkv_kernel,
        grid=grid,
        in_specs=[
            pl.BlockSpec((ROWB, D_IN), lambda i: (i, 0)),
            pl.BlockSpec((D_IN, D), lambda i: (0, 0)),
            pl.BlockSpec((1, D), lambda i: (0, 0)),
            pl.BlockSpec((D, 3 * D), lambda i: (0, 0)),
            pl.BlockSpec((1, 3 * D), lambda i: (0, 0)),
        ],
        out_specs=[
            pl.BlockSpec((ROWB, D), lambda i: (i, 0)),
            pl.BlockSpec((ROWB, D), lambda i: (i, 0)),
            pl.BlockSpec((4, ROWB, DV), lambda i: (0, i, 0)),
        ],
        out_shape=[
            jax.ShapeDtypeStruct((N, D), jnp.float32),
            jax.ShapeDtypeStruct((N, D), jnp.float32),
            jax.ShapeDtypeStruct((4, N, DV), jnp.float32),
        ],
    )(x, W_embed, b_embed.reshape(1, D), Wqkv, bqkv.reshape(1, 3 * D))


def _scores_kernel(q_ref, k_ref, s_ref):
    s_ref[0] = lax.dot_general(
        q_ref[...], k_ref[...], (((1,), (1,)), ((), ())),
        preferred_element_type=jnp.float32, precision=lax.Precision.DEFAULT)


SBM = 1024
SGM = (N + SBM - 1) // SBM


def _scores(qs, k):
    return pl.pallas_call(
        _scores_kernel,
        grid=(SGM, SGM),
        in_specs=[
            pl.BlockSpec((SBM, D), lambda i, j: (i, 0)),
            pl.BlockSpec((SBM, D), lambda i, j: (j, 0)),
        ],
        out_specs=pl.BlockSpec((1, SBM, SBM), lambda i, j: (i * SGM + j, 0, 0)),
        out_shape=jax.ShapeDtypeStruct((SGM * SGM, SBM, SBM), jnp.float32),
    )(qs, k)


def _agg_from_acc(acc):
    den = acc[0][:, QW:QW + 1]
    agg = jnp.concatenate([acc[i][:, :QW] for i in range(4)], axis=1)
    return agg / (den + 1e-16)


def _proj_qkv_kernel(acc_ref, wo_ref, bo_ref, wq_ref, bq_ref,
                     qs_ref, k_ref, vaug_ref):
    agg = _agg_from_acc(acc_ref)
    t = _dot(agg, wo_ref[...]) + bo_ref[...]
    t = jnp.maximum(t, 0.0)
    qkv = _dot(t, wq_ref[...]) + bq_ref[...]
    ones = jnp.ones((acc_ref.shape[1], 16), jnp.float32)
    q, k, vaug = _split_vaug(qkv, ones)
    qs_ref[...] = q
    k_ref[...] = k
    vaug_ref[...] = vaug


def _proj_qkv(acc, Wo, bo, Wqkv, bqkv):
    grid = (N // ROWB,)
    return pl.pallas_call(
        _proj_qkv_kernel,
        grid=grid,
        in_specs=[
            pl.BlockSpec((4, ROWB, DV), lambda i: (0, i, 0)),
            pl.BlockSpec((D, D), lambda i: (0, 0)),
            pl.BlockSpec((1, D), lambda i: (0, 0)),
            pl.BlockSpec((D, 3 * D), lambda i: (0, 0)),
            pl.BlockSpec((1, 3 * D), lambda i: (0, 0)),
        ],
        out_specs=[
            pl.BlockSpec((ROWB, D), lambda i: (i, 0)),
            pl.BlockSpec((ROWB, D), lambda i: (i, 0)),
            pl.BlockSpec((4, ROWB, DV), lambda i: (0, i, 0)),
        ],
        out_shape=[
            jax.ShapeDtypeStruct((N, D), jnp.float32),
            jax.ShapeDtypeStruct((N, D), jnp.float32),
            jax.ShapeDtypeStruct((4, N, DV), jnp.float32),
        ],
    )(acc, Wo, bo.reshape(1, D), Wqkv, bqkv.reshape(1, 3 * D))


def _final_kernel(acc_ref, wo_ref, bo_ref, wl_ref, bl_ref, o_ref):
    agg = _agg_from_acc(acc_ref)
    t = _dot(agg, wo_ref[...]) + bo_ref[...]
    logits = _dot(t, wl_ref[...]) + bl_ref[...]
    m = jnp.max(logits, axis=1, keepdims=True)
    lse = jnp.log(jnp.sum(jnp.exp(logits - m), axis=1, keepdims=True)) + m
    o_ref[...] = logits - lse


def _final(acc, Wo, bo, W_lin, b_lin):
    grid = (N // ROWB,)
    return pl.pallas_call(
        _final_kernel,
        grid=grid,
        in_specs=[
            pl.BlockSpec((4, ROWB, DV), lambda i: (0, i, 0)),
            pl.BlockSpec((D, D), lambda i: (0, 0)),
            pl.BlockSpec((1, D), lambda i: (0, 0)),
            pl.BlockSpec((D, C), lambda i: (0, 0)),
            pl.BlockSpec((1, C), lambda i: (0, 0)),
        ],
        out_specs=pl.BlockSpec((ROWB, C), lambda i: (i, 0)),
        out_shape=jax.ShapeDtypeStruct((N, C), jnp.float32),
    )(acc, Wo, bo.reshape(1, D), W_lin, b_lin.reshape(1, C))


NSC = 2
NSUB = 16
LANES = 16
CHUNK = 80
EPC = E // NSUB
NCHUNK = EPC // CHUNK
STRIPE = 624
TAIL = N - NSUB * STRIPE


def _edge_phase_sc(Sflat, vtab, src2d, dst2, sidx2, zz):
    mesh = plsc.VectorSubcoreMesh(core_axis_name="c", subcore_axis_name="s")
    cp = pltpu.CompilerParams()
    if "needs_layout_passes" in pltpu.CompilerParams.__dataclass_fields__:
        cp = dataclasses.replace(cp, needs_layout_passes=False)
    if "use_tc_tiling_on_sc" in pltpu.CompilerParams.__dataclass_fields__:
        cp = dataclasses.replace(cp, use_tc_tiling_on_sc=False)

    @functools.partial(
        pl.kernel, mesh=mesh, compiler_params=cp,
        out_type=jax.ShapeDtypeStruct((4 * N, DV), jnp.float32),
        scratch_types=[
            pltpu.VMEM((NCHUNK, CHUNK), jnp.int32),
            pltpu.VMEM((1, CHUNK), jnp.int32),
            pltpu.VMEM((1, CHUNK), jnp.int32),
            pltpu.VMEM((1, CHUNK), jnp.int32),
            pltpu.VMEM((1, CHUNK), jnp.int32),
            pltpu.VMEM((CHUNK,), jnp.float32),
            pltpu.VMEM((CHUNK,), jnp.float32),
            pltpu.VMEM((CHUNK, DV), jnp.float32),
            pltpu.VMEM((CHUNK, DV), jnp.float32),
            pltpu.VMEM_SHARED((N, DV), jnp.float32),
            pltpu.SemaphoreType.DMA,
            pltpu.SemaphoreType.DMA,
            pltpu.SemaphoreType.DMA,
            pltpu.SemaphoreType.DMA,
            pltpu.SemaphoreType.DMA,
            pltpu.SemaphoreType.DMA,
        ],
    )
    def ker(sflat_hbm, vtab_hbm, src2d_hbm, dst2_hbm, sidx2_hbm, zz_hbm,
            out_hbm, dst_v, sia, sib, sra, srb, wa, wb, rowsa, rowsb, acc,
            isa_, isb_, gsa, gsb, ssa, ssb):
        core = lax.axis_index("c")
        sid = lax.axis_index("s")

        pltpu.sync_copy(dst2_hbm.at[pl.ds(sid * NCHUNK, NCHUNK)], dst_v)

        NT = NCHUNK // 2
        for p in range(2):
            q = 2 * core + p
            sbase = sid * NCHUNK
            rbase = (q * NSUB + sid) * NCHUNK

            def issue_idx(si_buf, sr_buf, isem, i):
                pltpu.async_copy(sidx2_hbm.at[pl.ds(sbase + i, 1)],
                                 si_buf, isem)
                pltpu.async_copy(src2d_hbm.at[pl.ds(rbase + i, 1)],
                                 sr_buf, isem)

            def wait_idx(si_buf, sr_buf, isem, i):
                pltpu.make_async_copy(sidx2_hbm.at[pl.ds(sbase + i, 1)],
                                      si_buf, isem).wait()
                pltpu.make_async_copy(src2d_hbm.at[pl.ds(rbase + i, 1)],
                                      sr_buf, isem).wait()

            def issue_gathers(si_buf, sr_buf, w_buf, rows_buf, gsem):
                pltpu.async_copy(sflat_hbm.at[si_buf.at[0]], w_buf, gsem)
                pltpu.async_copy(vtab_hbm.at[sr_buf.at[0]], rows_buf, gsem)

            def wait_gathers(si_buf, sr_buf, w_buf, rows_buf, gsem):
                pltpu.make_async_copy(sflat_hbm.at[si_buf.at[0]], w_buf,
                                      gsem).wait()
                pltpu.make_async_copy(vtab_hbm.at[sr_buf.at[0]], rows_buf,
                                      gsem).wait()

            def compute(w_buf, rows_buf):
                for c in range(CHUNK // LANES):
                    sl = pl.ds(c * LANES, LANES)
                    w_buf[sl] = jnp.exp(w_buf[sl])

                @pl.loop(0, CHUNK)
                def _(j):
                    wspl = plsc.load_gather(
                        w_buf, [jnp.full((LANES,), j, jnp.int32)])
                    for c in range(DV // LANES):
                        sl = pl.ds(c * LANES, LANES)
                        rows_buf[j, sl] = rows_buf[j, sl] * wspl

            def issue_scatter(rows_buf, ssem, i):
                pltpu.async_copy(rows_buf, acc.at[dst_v.at[i]], ssem,
                                 add=True)

            def wait_scatter(rows_buf, ssem, i):
                pltpu.make_async_copy(rows_buf, acc.at[dst_v.at[i]],
                                      ssem).wait()

            pltpu.sync_copy(zz_hbm, acc.at[pl.ds(sid * STRIPE, STRIPE)])

            @pl.when(sid == NSUB - 1)
            def _():
                pltpu.sync_copy(zz_hbm.at[pl.ds(0, TAIL)],
                                acc.at[pl.ds(NSUB * STRIPE, TAIL)])

            plsc.subcore_barrier()

            issue_idx(sia, sra, isa_, 0)
            issue_idx(sib, srb, isb_, 1)
            wait_idx(sia, sra, isa_, 0)
            issue_gathers(sia, sra, wa, rowsa, gsa)

            @pl.loop(0, NT)
            def _(t):
                i0 = 2 * t
                i1 = i0 + 1

                @pl.when(t > 0)
                def _():
                    wait_scatter(rowsb, ssb, i1 - 2)

                wait_idx(sib, srb, isb_, i1)
                issue_gathers(sib, srb, wb, rowsb, gsb)
                wait_gathers(sia, sra, wa, rowsa, gsa)

                @pl.when(t < NT - 1)
                def _():
                    issue_idx(sia, sra, isa_, i0 + 2)

                compute(wa, rowsa)
                issue_scatter(rowsa, ssa, i0)

                @pl.when(t < NT - 1)
                def _():
                    wait_scatter(rowsa, ssa, i0)
                    wait_idx(sia, sra, isa_, i0 + 2)
                    issue_gathers(sia, sra, wa, rowsa, gsa)

                wait_gathers(sib, srb, wb, rowsb, gsb)

                @pl.when(t < NT - 1)
                def _():
                    issue_idx(sib, srb, isb_, i1 + 2)

                compute(wb, rowsb)
                issue_scatter(rowsb, ssb, i1)

            wait_scatter(rowsa, ssa, NCHUNK - 2)
            wait_scatter(rowsb, ssb, NCHUNK - 1)
            plsc.subcore_barrier()
            pltpu.sync_copy(acc.at[pl.ds(sid * STRIPE, STRIPE)],
                            out_hbm.at[pl.ds(q * N + sid * STRIPE, STRIPE)])

            @pl.when(sid == NSUB - 1)
            def _():
                pltpu.sync_copy(
                    acc.at[pl.ds(NSUB * STRIPE, TAIL)],
                    out_hbm.at[pl.ds(q * N + NSUB * STRIPE, TAIL)])

    return ker(Sflat, vtab, src2d, dst2, sidx2, zz)


def _edge_phase(S, vaug, src2d, dst2, sidx2, zz):
    acc = _edge_phase_sc(S.reshape(-1), vaug.reshape(4 * N, DV),
                         src2d, dst2, sidx2, zz)
    return acc.reshape(4, N, DV)


def kernel(x, edge_index, W_embed, b_embed, Wqkv1, bqkv1, Wo1, bo1,
           Wqkv2, bqkv2, Wo2, bo2, W_lin, b_lin):
    src = edge_index[0].astype(jnp.int32)
    dst = edge_index[1].astype(jnp.int32)
    sidx = (((dst >> 10) * SGM + (src >> 10)) << 20) \
        + ((dst & (SBM - 1)) << 10) + (src & (SBM - 1))
    src4 = jnp.concatenate([src, src + N, src + 2 * N, src + 3 * N])
    src2d = src4.reshape(-1, CHUNK)
    dst2 = dst.reshape(-1, CHUNK)
    sidx2 = sidx.reshape(-1, CHUNK)
    zz = jnp.zeros((STRIPE, DV), jnp.float32)

    qs, k, vaug = _embed_qkv(x, W_embed, b_embed, Wqkv1, bqkv1)
    S = _scores(qs, k)
    acc = _edge_phase(S, vaug, src2d, dst2, sidx2, zz)
    qs2, k2, vaug2 = _proj_qkv(acc, Wo1, bo1, Wqkv2, bqkv2)
    S2 = _scores(qs2, k2)
    acc2 = _edge_phase(S2, vaug2, src2d, dst2, sidx2, zz)
    return _final(acc2, Wo2, bo2, W_lin, b_lin)
